# baseline (speedup 1.0000x reference)
import functools

import jax
import jax.numpy as jnp
from jax import lax
from jax.experimental import pallas as pl
from jax.experimental.pallas import tpu as pltpu
from jax.experimental.pallas import tpu_sc as plsc

BATCH = 16384
N_USERS = 1500
N_ITEMS = 2000
DIM = 3

_INTERP = False
_INFO = plsc.get_sparse_core_info()
_NC = _INFO.num_cores
_NS = _INFO.num_subcores
_L = _INFO.num_lanes
_NW = _NC * _NS
_BPW = BATCH // _NW
_NVEC = _BPW // _L

_TBL = (N_USERS + N_ITEMS) * DIM
_CH = 664
_TPAD = _CH * 16


def _sc_body(data_hbm, tbl_hbm, out_hbm, t_s, t_v, ui_v, vi_v, out_v,
             sem, sem2):
    wid = lax.axis_index("s") * _NC + lax.axis_index("c")
    sid = lax.axis_index("s")
    base = wid * _BPW

    c3 = pltpu.async_copy(data_hbm.at[0, pl.ds(base, _BPW)], ui_v, sem2)
    c4 = pltpu.async_copy(data_hbm.at[1, pl.ds(base, _BPW)], vi_v, sem2)
    cc = pltpu.async_copy(tbl_hbm.at[pl.ds(sid * _CH, _CH)],
                          t_v.at[pl.ds(sid * _CH, _CH)], sem)
    cc.wait()
    cs = pltpu.async_copy(t_v.at[pl.ds(sid * _CH, _CH)],
                          t_s.at[pl.ds(sid * _CH, _CH)], sem)
    cs.wait()
    plsc.subcore_barrier()
    ct = pltpu.async_copy(t_s, t_v, sem)
    ct.wait()
    c3.wait()
    c4.wait()

    dim = jnp.full((_L,), DIM, jnp.int32)
    voff = jnp.full((_L,), N_USERS * DIM, jnp.int32)
    one = jnp.full((_L,), 1, jnp.int32)

    def step(i, _):
        off = pl.multiple_of(i * _L, _L)
        ui = ui_v[pl.ds(off, _L)] * dim
        vi = vi_v[pl.ds(off, _L)] * dim + voff
        acc = plsc.load_gather(t_v, [ui]) * plsc.load_gather(t_v, [vi])
        for _c in range(1, DIM):
            ui = ui + one
            vi = vi + one
            acc += plsc.load_gather(t_v, [ui]) * plsc.load_gather(t_v, [vi])
        out_v[pl.ds(off, _L)] = acc
        return ()

    lax.fori_loop(0, _NVEC, step, (), unroll=_NVEC)

    pltpu.sync_copy(out_v, out_hbm.at[pl.ds(base, _BPW)])


_sc_kernel = functools.partial(
    pl.kernel,
    out_type=jax.ShapeDtypeStruct((BATCH,), jnp.float32),
    mesh=plsc.VectorSubcoreMesh(core_axis_name="c", subcore_axis_name="s"),
    compiler_params=pltpu.CompilerParams(needs_layout_passes=False),
    interpret=_INTERP,
    scratch_types=[
        pltpu.VMEM_SHARED((_TPAD,), jnp.float32),
        pltpu.VMEM((_TPAD,), jnp.float32),
        pltpu.VMEM((_BPW,), jnp.int32),
        pltpu.VMEM((_BPW,), jnp.int32),
        pltpu.VMEM((_BPW,), jnp.float32),
        pltpu.SemaphoreType.DMA,
        pltpu.SemaphoreType.DMA,
    ],
)(_sc_body)


@jax.jit
def kernel(data, user_factors, item_factors):
    tbl = jnp.concatenate(
        [user_factors, item_factors], axis=0).reshape(-1)
    tbl = jnp.pad(tbl, (0, _TPAD - _TBL))
    return _sc_kernel(data.astype(jnp.int32), tbl)


# speedup vs baseline: 1.0089x; 1.0089x over previous
import functools

import jax
import jax.numpy as jnp
from jax import lax
from jax.experimental import pallas as pl
from jax.experimental.pallas import tpu as pltpu
from jax.experimental.pallas import tpu_sc as plsc

BATCH = 16384
N_USERS = 1500
N_ITEMS = 2000
DIM = 3

_INTERP = False
_INFO = plsc.get_sparse_core_info()
_NC = _INFO.num_cores
_NS = _INFO.num_subcores
_L = _INFO.num_lanes
_NW = _NC * _NS
_BPW = BATCH // _NW
_NVEC = _BPW // _L

_TBL = (N_USERS + N_ITEMS) * DIM
_CH = 664
_TPAD = _CH * 16


def _sc_body(data_hbm, tbl_hbm, out_hbm, t_s, t_v, ui_v, vi_v, out_v,
             sem, sem2):
    wid = lax.axis_index("s") * _NC + lax.axis_index("c")
    sid = lax.axis_index("s")
    base = wid * _BPW

    c3 = pltpu.async_copy(data_hbm.at[0, pl.ds(base, _BPW)], ui_v, sem2)
    c4 = pltpu.async_copy(data_hbm.at[1, pl.ds(base, _BPW)], vi_v, sem2)
    cc = pltpu.async_copy(tbl_hbm.at[pl.ds(sid * _CH, _CH)],
                          t_v.at[pl.ds(sid * _CH, _CH)], sem)
    cc.wait()
    cs = pltpu.async_copy(t_v.at[pl.ds(sid * _CH, _CH)],
                          t_s.at[pl.ds(sid * _CH, _CH)], sem)
    cs.wait()
    plsc.subcore_barrier()
    ct = pltpu.async_copy(t_s, t_v, sem)
    ct.wait()
    c3.wait()
    c4.wait()

    dim = jnp.full((_L,), DIM, jnp.int32)
    voff = jnp.full((_L,), N_USERS * DIM, jnp.int32)
    one = jnp.full((_L,), 1, jnp.int32)

    def step(i, _):
        off = pl.multiple_of(i * _L, _L)
        ui = ui_v[pl.ds(off, _L)] * dim
        vi = vi_v[pl.ds(off, _L)] * dim + voff
        acc = plsc.load_gather(t_v, [ui]) * plsc.load_gather(t_v, [vi])
        for _c in range(1, DIM):
            ui = ui + one
            vi = vi + one
            acc += plsc.load_gather(t_v, [ui]) * plsc.load_gather(t_v, [vi])
        out_v[pl.ds(off, _L)] = acc
        return ()

    lax.fori_loop(0, _NVEC, step, (), unroll=8)

    pltpu.sync_copy(out_v, out_hbm.at[pl.ds(base, _BPW)])


_sc_kernel = functools.partial(
    pl.kernel,
    out_type=jax.ShapeDtypeStruct((BATCH,), jnp.float32),
    mesh=plsc.VectorSubcoreMesh(core_axis_name="c", subcore_axis_name="s"),
    compiler_params=pltpu.CompilerParams(needs_layout_passes=False),
    interpret=_INTERP,
    scratch_types=[
        pltpu.VMEM_SHARED((_TPAD,), jnp.float32),
        pltpu.VMEM((_TPAD,), jnp.float32),
        pltpu.VMEM((_BPW,), jnp.int32),
        pltpu.VMEM((_BPW,), jnp.int32),
        pltpu.VMEM((_BPW,), jnp.float32),
        pltpu.SemaphoreType.DMA,
        pltpu.SemaphoreType.DMA,
    ],
)(_sc_body)


@jax.jit
def kernel(data, user_factors, item_factors):
    tbl = jnp.concatenate(
        [user_factors, item_factors], axis=0).reshape(-1)
    tbl = jnp.pad(tbl, (0, _TPAD - _TBL))
    return _sc_kernel(data.astype(jnp.int32), tbl)


# speedup vs baseline: 1.0154x; 1.0064x over previous
import functools

import jax
import jax.numpy as jnp
from jax import lax
from jax.experimental import pallas as pl
from jax.experimental.pallas import tpu as pltpu
from jax.experimental.pallas import tpu_sc as plsc

BATCH = 16384
N_USERS = 1500
N_ITEMS = 2000
DIM = 3

_INTERP = False
_INFO = plsc.get_sparse_core_info()
_NC = _INFO.num_cores
_NS = _INFO.num_subcores
_L = _INFO.num_lanes
_NW = _NC * _NS
_BPW = BATCH // _NW
_NVEC = _BPW // _L

_TBL = (N_USERS + N_ITEMS) * DIM
_CH = 664
_TPAD = _CH * 16


def _sc_body(data_hbm, tbl_hbm, out_hbm, t_s, t_v, ui_v, vi_v, out_v,
             sem, sem2):
    wid = lax.axis_index("s") * _NC + lax.axis_index("c")
    sid = lax.axis_index("s")
    base = wid * _BPW

    c3 = pltpu.async_copy(data_hbm.at[0, pl.ds(base, _BPW)], ui_v, sem2)
    c4 = pltpu.async_copy(data_hbm.at[1, pl.ds(base, _BPW)], vi_v, sem2)
    cc = pltpu.async_copy(tbl_hbm.at[pl.ds(sid * _CH, _CH)],
                          t_v.at[pl.ds(sid * _CH, _CH)], sem)
    cc.wait()
    cs = pltpu.async_copy(t_v.at[pl.ds(sid * _CH, _CH)],
                          t_s.at[pl.ds(sid * _CH, _CH)], sem)
    cs.wait()
    plsc.subcore_barrier()
    ct = pltpu.async_copy(t_s, t_v, sem)
    ct.wait()
    c3.wait()
    c4.wait()

    dim = jnp.full((_L,), DIM, jnp.int32)
    voff = jnp.full((_L,), N_USERS * DIM, jnp.int32)
    one = jnp.full((_L,), 1, jnp.int32)

    def step(i, _):
        off = pl.multiple_of(i * _L, _L)
        ui = ui_v[pl.ds(off, _L)] * dim
        vi = vi_v[pl.ds(off, _L)] * dim + voff
        acc = plsc.load_gather(t_v, [ui]) * plsc.load_gather(t_v, [vi])
        for _c in range(1, DIM):
            ui = ui + one
            vi = vi + one
            acc += plsc.load_gather(t_v, [ui]) * plsc.load_gather(t_v, [vi])
        out_v[pl.ds(off, _L)] = acc
        return ()

    lax.fori_loop(0, _NVEC, step, (), unroll=4)

    pltpu.sync_copy(out_v, out_hbm.at[pl.ds(base, _BPW)])


_sc_kernel = functools.partial(
    pl.kernel,
    out_type=jax.ShapeDtypeStruct((BATCH,), jnp.float32),
    mesh=plsc.VectorSubcoreMesh(core_axis_name="c", subcore_axis_name="s"),
    compiler_params=pltpu.CompilerParams(needs_layout_passes=False),
    interpret=_INTERP,
    scratch_types=[
        pltpu.VMEM_SHARED((_TPAD,), jnp.float32),
        pltpu.VMEM((_TPAD,), jnp.float32),
        pltpu.VMEM((_BPW,), jnp.int32),
        pltpu.VMEM((_BPW,), jnp.int32),
        pltpu.VMEM((_BPW,), jnp.float32),
        pltpu.SemaphoreType.DMA,
        pltpu.SemaphoreType.DMA,
    ],
)(_sc_body)


@jax.jit
def kernel(data, user_factors, item_factors):
    tbl = jnp.concatenate(
        [user_factors, item_factors], axis=0).reshape(-1)
    tbl = jnp.pad(tbl, (0, _TPAD - _TBL))
    return _sc_kernel(data.astype(jnp.int32), tbl)


# speedup vs baseline: 1.0217x; 1.0062x over previous
import functools

import jax
import jax.numpy as jnp
from jax import lax
from jax.experimental import pallas as pl
from jax.experimental.pallas import tpu as pltpu
from jax.experimental.pallas import tpu_sc as plsc

BATCH = 16384
N_USERS = 1500
N_ITEMS = 2000
DIM = 3

_INTERP = False
_INFO = plsc.get_sparse_core_info()
_NC = _INFO.num_cores
_NS = _INFO.num_subcores
_L = _INFO.num_lanes
_NW = _NC * _NS
_BPW = BATCH // _NW
_NVEC = _BPW // _L

_TBL = (N_USERS + N_ITEMS) * DIM
_CH = 664
_TPAD = _CH * 16


def _sc_body(data_hbm, tbl_hbm, out_hbm, t_s, t_v, ui_v, vi_v, out_v,
             sem, sem2):
    wid = lax.axis_index("s") * _NC + lax.axis_index("c")
    sid = lax.axis_index("s")
    base = wid * _BPW

    c3 = pltpu.async_copy(data_hbm.at[0, pl.ds(base, _BPW)], ui_v, sem2)
    c4 = pltpu.async_copy(data_hbm.at[1, pl.ds(base, _BPW)], vi_v, sem2)
    cc = pltpu.async_copy(tbl_hbm.at[pl.ds(sid * _CH, _CH)],
                          t_v.at[pl.ds(sid * _CH, _CH)], sem)
    cc.wait()
    cs = pltpu.async_copy(t_v.at[pl.ds(sid * _CH, _CH)],
                          t_s.at[pl.ds(sid * _CH, _CH)], sem)
    cs.wait()
    plsc.subcore_barrier()
    ct = pltpu.async_copy(t_s, t_v, sem)
    ct.wait()
    c3.wait()
    c4.wait()

    dim = jnp.full((_L,), DIM, jnp.int32)
    voff = jnp.full((_L,), N_USERS * DIM, jnp.int32)
    one = jnp.full((_L,), 1, jnp.int32)

    def step(i, _):
        off = pl.multiple_of(i * _L, _L)
        ui = ui_v[pl.ds(off, _L)] * dim
        vi = vi_v[pl.ds(off, _L)] * dim + voff
        acc = plsc.load_gather(t_v, [ui]) * plsc.load_gather(t_v, [vi])
        for _c in range(1, DIM):
            ui = ui + one
            vi = vi + one
            acc += plsc.load_gather(t_v, [ui]) * plsc.load_gather(t_v, [vi])
        out_v[pl.ds(off, _L)] = acc
        return ()

    lax.fori_loop(0, _NVEC, step, (), unroll=2)

    pltpu.sync_copy(out_v, out_hbm.at[pl.ds(base, _BPW)])


_sc_kernel = functools.partial(
    pl.kernel,
    out_type=jax.ShapeDtypeStruct((BATCH,), jnp.float32),
    mesh=plsc.VectorSubcoreMesh(core_axis_name="c", subcore_axis_name="s"),
    compiler_params=pltpu.CompilerParams(needs_layout_passes=False),
    interpret=_INTERP,
    scratch_types=[
        pltpu.VMEM_SHARED((_TPAD,), jnp.float32),
        pltpu.VMEM((_TPAD,), jnp.float32),
        pltpu.VMEM((_BPW,), jnp.int32),
        pltpu.VMEM((_BPW,), jnp.int32),
        pltpu.VMEM((_BPW,), jnp.float32),
        pltpu.SemaphoreType.DMA,
        pltpu.SemaphoreType.DMA,
    ],
)(_sc_body)


@jax.jit
def kernel(data, user_factors, item_factors):
    tbl = jnp.concatenate(
        [user_factors, item_factors], axis=0).reshape(-1)
    tbl = jnp.pad(tbl, (0, _TPAD - _TBL))
    return _sc_kernel(data.astype(jnp.int32), tbl)


# speedup vs baseline: 1.0298x; 1.0079x over previous
import functools

import jax
import jax.numpy as jnp
from jax import lax
from jax.experimental import pallas as pl
from jax.experimental.pallas import tpu as pltpu
from jax.experimental.pallas import tpu_sc as plsc

BATCH = 16384
N_USERS = 1500
N_ITEMS = 2000
DIM = 3

_INTERP = False
_INFO = plsc.get_sparse_core_info()
_NC = _INFO.num_cores
_NS = _INFO.num_subcores
_L = _INFO.num_lanes
_NW = _NC * _NS
_BPW = BATCH // _NW
_NVEC = _BPW // _L

_TBL = (N_USERS + N_ITEMS) * DIM
_CH = 664
_TPAD = _CH * 16


def _sc_body(data_hbm, tbl_hbm, out_hbm, t_s, t_v, ui_v, vi_v, out_v,
             sem, sem2):
    wid = lax.axis_index("s") * _NC + lax.axis_index("c")
    sid = lax.axis_index("s")
    base = wid * _BPW

    c3 = pltpu.async_copy(data_hbm.at[0, pl.ds(base, _BPW)], ui_v, sem2)
    c4 = pltpu.async_copy(data_hbm.at[1, pl.ds(base, _BPW)], vi_v, sem2)
    cc = pltpu.async_copy(tbl_hbm.at[pl.ds(sid * _CH, _CH)],
                          t_v.at[pl.ds(sid * _CH, _CH)], sem)
    cc.wait()
    cs = pltpu.async_copy(t_v.at[pl.ds(sid * _CH, _CH)],
                          t_s.at[pl.ds(sid * _CH, _CH)], sem)
    cs.wait()
    plsc.subcore_barrier()
    ct = pltpu.async_copy(t_s, t_v, sem)
    ct.wait()
    c3.wait()
    c4.wait()

    dim = jnp.full((_L,), DIM, jnp.int32)
    voff = jnp.full((_L,), N_USERS * DIM, jnp.int32)
    one = jnp.full((_L,), 1, jnp.int32)

    def step(i, _):
        off = pl.multiple_of(i * _L, _L)
        ui = ui_v[pl.ds(off, _L)] * dim
        vi = vi_v[pl.ds(off, _L)] * dim + voff
        acc = plsc.load_gather(t_v, [ui]) * plsc.load_gather(t_v, [vi])
        for _c in range(1, DIM):
            ui = ui + one
            vi = vi + one
            acc += plsc.load_gather(t_v, [ui]) * plsc.load_gather(t_v, [vi])
        out_v[pl.ds(off, _L)] = acc
        return ()

    lax.fori_loop(0, _NVEC, step, ())

    pltpu.sync_copy(out_v, out_hbm.at[pl.ds(base, _BPW)])


_sc_kernel = functools.partial(
    pl.kernel,
    out_type=jax.ShapeDtypeStruct((BATCH,), jnp.float32),
    mesh=plsc.VectorSubcoreMesh(core_axis_name="c", subcore_axis_name="s"),
    compiler_params=pltpu.CompilerParams(needs_layout_passes=False),
    interpret=_INTERP,
    scratch_types=[
        pltpu.VMEM_SHARED((_TPAD,), jnp.float32),
        pltpu.VMEM((_TPAD,), jnp.float32),
        pltpu.VMEM((_BPW,), jnp.int32),
        pltpu.VMEM((_BPW,), jnp.int32),
        pltpu.VMEM((_BPW,), jnp.float32),
        pltpu.SemaphoreType.DMA,
        pltpu.SemaphoreType.DMA,
    ],
)(_sc_body)


@jax.jit
def kernel(data, user_factors, item_factors):
    tbl = jnp.concatenate(
        [user_factors, item_factors], axis=0).reshape(-1)
    tbl = jnp.pad(tbl, (0, _TPAD - _TBL))
    return _sc_kernel(data.astype(jnp.int32), tbl)
